# baseline (speedup 1.0000x reference)
import functools

import jax
import jax.numpy as jnp
from jax import lax
from jax.experimental import pallas as pl
from jax.experimental.pallas import tpu as pltpu
from jax.experimental.pallas import tpu_sc as plsc

N = 1_600_000
G = 64
NUM_WORKERS = 32
PER = N // NUM_WORKERS
LANES = 16
LR = 0.01
NCHUNK = 5
CHUNK = PER // NCHUNK
UNROLL = 5

@functools.cache
def _make_segment_partials():
    mesh = plsc.VectorSubcoreMesh(core_axis_name="c", subcore_axis_name="s",
                                  num_cores=2, num_subcores=16)
    return pl.kernel(
        _segment_partials_body,
        mesh=mesh,
        compiler_params=pltpu.CompilerParams(use_tc_tiling_on_sc=False,
                                             needs_layout_passes=False),
        out_type=[
            jax.ShapeDtypeStruct((NUM_WORKERS, LANES * G), jnp.float32),
            jax.ShapeDtypeStruct((NUM_WORKERS, LANES * G), jnp.float32),
        ],
        scratch_types=[
            pltpu.VMEM((CHUNK,), jnp.float32),
            pltpu.VMEM((CHUNK,), jnp.float32),
            pltpu.VMEM((CHUNK,), jnp.int32),
            pltpu.VMEM((CHUNK,), jnp.int32),
            pltpu.VMEM((LANES * G,), jnp.float32),
            pltpu.VMEM((LANES * G,), jnp.float32),
            pltpu.VMEM((LANES * G,), jnp.float32),
            pltpu.VMEM((LANES * G,), jnp.float32),
            pltpu.SemaphoreType.DMA,
            pltpu.SemaphoreType.DMA,
            pltpu.SemaphoreType.DMA,
            pltpu.SemaphoreType.DMA,
        ],
    )


def _segment_partials_body(loss_hbm, groups_hbm, sums_out, cnts_out,
                           la, lb, ga, gb, sacc, cacc, sacc1, cacc1,
                           sla, slb, sga, sgb):
    wid = lax.axis_index("s") * 2 + lax.axis_index("c")
    base = wid * PER
    lbufs = (la, lb)
    gbufs = (ga, gb)
    lsems = (sla, slb)
    gsems = (sga, sgb)

    def start(c):
        b = c % 2
        src = pl.ds(base + c * CHUNK, CHUNK)
        return (
            pltpu.async_copy(loss_hbm.at[src], lbufs[b], lsems[b]),
            pltpu.async_copy(groups_hbm.at[src], gbufs[b], gsems[b]),
        )

    pending = {0: start(0)}

    zero = jnp.zeros((LANES,), jnp.float32)
    for k in range(G):
        sacc[pl.ds(k * LANES, LANES)] = zero
        cacc[pl.ds(k * LANES, LANES)] = zero
        sacc1[pl.ds(k * LANES, LANES)] = zero
        cacc1[pl.ds(k * LANES, LANES)] = zero

    lane_base = lax.iota(jnp.int32, LANES) * G
    ones = jnp.full((LANES,), 1.0, jnp.float32)

    for c in range(NCHUNK):
        if c + 1 < NCHUNK:
            pending[c + 1] = start(c + 1)
        cpl, cpg = pending.pop(c)
        cpl.wait()
        cpg.wait()
        lB = lbufs[c % 2]
        gB = gbufs[c % 2]

        def body(i, carry):
            s0 = i * (LANES * UNROLL)
            for u in range(UNROLL):
                s = s0 + u * LANES
                g = gB[pl.ds(s, LANES)]
                lv = lB[pl.ds(s, LANES)]
                idx = lane_base + g
                sb = sacc if u % 2 == 0 else sacc1
                cb = cacc if u % 2 == 0 else cacc1
                plsc.addupdate_scatter(sb, [idx], lv)
                plsc.addupdate_scatter(cb, [idx], ones)
            return carry

        lax.fori_loop(0, CHUNK // (LANES * UNROLL), body, 0)

    for k in range(G):
        d = pl.ds(k * LANES, LANES)
        sacc[d] = sacc[d] + sacc1[d]
        cacc[d] = cacc[d] + cacc1[d]

    pltpu.sync_copy(sacc, sums_out.at[wid])
    pltpu.sync_copy(cacc, cnts_out.at[wid])


def _finalize(sums_ref, cnts_ref, gw_ref, out_ref):
    gs_rows = sums_ref[:, 0:G]
    gc_rows = cnts_ref[:, 0:G]
    for l in range(1, LANES):
        gs_rows = gs_rows + sums_ref[:, l * G:(l + 1) * G]
        gc_rows = gc_rows + cnts_ref[:, l * G:(l + 1) * G]
    gs = jnp.sum(gs_rows, axis=0, keepdims=True)
    gc = jnp.sum(gc_rows, axis=0, keepdims=True)
    gl = jnp.where(gc > 0, gs / jnp.maximum(gc, 1.0), 0.0)
    w = gw_ref[...] * jnp.exp(LR * gl)
    nrm = jnp.sqrt(jnp.sum(w * w))
    r = jnp.sum(gl * w) / jnp.maximum(nrm, 1e-12)
    out_ref[...] = jnp.broadcast_to(r, (1, 1))


@jax.jit
def kernel(loss, groups, group_weights):
    sums, cnts = _make_segment_partials()(loss, groups)
    out = pl.pallas_call(
        _finalize,
        out_shape=jax.ShapeDtypeStruct((1, 1), jnp.float32),
    )(sums, cnts, group_weights.reshape(1, G))
    return out[0, 0]


# speedup vs baseline: 1.0742x; 1.0742x over previous
import functools

import jax
import jax.numpy as jnp
from jax import lax
from jax.experimental import pallas as pl
from jax.experimental.pallas import tpu as pltpu
from jax.experimental.pallas import tpu_sc as plsc

N = 1_600_000
G = 64
NUM_WORKERS = 32
PER = N // NUM_WORKERS
LANES = 16
LR = 0.01
NCHUNK = 5
CHUNK = PER // NCHUNK
UNROLL = 5

@functools.cache
def _make_segment_partials():
    mesh = plsc.VectorSubcoreMesh(core_axis_name="c", subcore_axis_name="s",
                                  num_cores=2, num_subcores=16)
    return pl.kernel(
        _segment_partials_body,
        mesh=mesh,
        compiler_params=pltpu.CompilerParams(use_tc_tiling_on_sc=False,
                                             needs_layout_passes=False),
        out_type=[
            jax.ShapeDtypeStruct((NUM_WORKERS, LANES * G), jnp.float32),
            jax.ShapeDtypeStruct((NUM_WORKERS, LANES * G), jnp.float32),
        ],
        scratch_types=[
            pltpu.VMEM((CHUNK,), jnp.float32),
            pltpu.VMEM((CHUNK,), jnp.float32),
            pltpu.VMEM((CHUNK,), jnp.int32),
            pltpu.VMEM((CHUNK,), jnp.int32),
            pltpu.VMEM((LANES * G,), jnp.float32),
            pltpu.VMEM((LANES * G,), jnp.float32),
            pltpu.SemaphoreType.DMA,
            pltpu.SemaphoreType.DMA,
            pltpu.SemaphoreType.DMA,
            pltpu.SemaphoreType.DMA,
        ],
    )


def _segment_partials_body(loss_hbm, groups_hbm, sums_out, cnts_out,
                           la, lb, ga, gb, sacc, cacc,
                           sla, slb, sga, sgb):
    wid = lax.axis_index("s") * 2 + lax.axis_index("c")
    base = wid * PER
    lbufs = (la, lb)
    gbufs = (ga, gb)
    lsems = (sla, slb)
    gsems = (sga, sgb)

    def start(c):
        b = c % 2
        src = pl.ds(base + c * CHUNK, CHUNK)
        return (
            pltpu.async_copy(loss_hbm.at[src], lbufs[b], lsems[b]),
            pltpu.async_copy(groups_hbm.at[src], gbufs[b], gsems[b]),
        )

    pending = {0: start(0)}

    zero = jnp.zeros((LANES,), jnp.float32)
    for k in range(G):
        sacc[pl.ds(k * LANES, LANES)] = zero
        cacc[pl.ds(k * LANES, LANES)] = zero

    lane = lax.iota(jnp.int32, LANES)
    ones = jnp.full((LANES,), 1.0, jnp.float32)

    for c in range(NCHUNK):
        if c + 1 < NCHUNK:
            pending[c + 1] = start(c + 1)
        cpl, cpg = pending.pop(c)
        cpl.wait()
        cpg.wait()
        lB = lbufs[c % 2]
        gB = gbufs[c % 2]

        def body(i, carry):
            s0 = i * (LANES * UNROLL)
            for u in range(UNROLL):
                s = s0 + u * LANES
                g = gB[pl.ds(s, LANES)]
                lv = lB[pl.ds(s, LANES)]
                idx = g * LANES + lane
                plsc.addupdate_scatter(sacc, [idx], lv)
                plsc.addupdate_scatter(cacc, [idx], ones)
            return carry

        lax.fori_loop(0, CHUNK // (LANES * UNROLL), body, 0)

    pltpu.sync_copy(sacc, sums_out.at[wid])
    pltpu.sync_copy(cacc, cnts_out.at[wid])


def _finalize(sums_ref, cnts_ref, gw_ref, out_ref):
    srow = jnp.sum(sums_ref[...], axis=0, keepdims=True)
    crow = jnp.sum(cnts_ref[...], axis=0, keepdims=True)
    col_g = lax.broadcasted_iota(jnp.int32, (G, LANES * G), 1) // LANES
    row_g = lax.broadcasted_iota(jnp.int32, (G, LANES * G), 0)
    mask = col_g == row_g
    gs = jnp.sum(jnp.where(mask, jnp.broadcast_to(srow, (G, LANES * G)), 0.0),
                 axis=1, keepdims=True)
    gc = jnp.sum(jnp.where(mask, jnp.broadcast_to(crow, (G, LANES * G)), 0.0),
                 axis=1, keepdims=True)
    gl = jnp.where(gc > 0, gs / jnp.maximum(gc, 1.0), 0.0)
    w = gw_ref[...] * jnp.exp(LR * gl)
    nrm = jnp.sqrt(jnp.sum(w * w))
    r = jnp.sum(gl * w) / jnp.maximum(nrm, 1e-12)
    out_ref[...] = jnp.broadcast_to(r, (1, 1))


@jax.jit
def kernel(loss, groups, group_weights):
    sums, cnts = _make_segment_partials()(loss, groups)
    out = pl.pallas_call(
        _finalize,
        out_shape=jax.ShapeDtypeStruct((1, 1), jnp.float32),
    )(sums, cnts, group_weights.reshape(G, 1))
    return out[0, 0]
